# baseline (speedup 1.0000x reference)
import functools

import jax
import jax.numpy as jnp
from jax import lax
from jax.experimental import pallas as pl
from jax.experimental.pallas import tpu as pltpu
from jax.experimental.pallas import tpu_sc as plsc

NC, NS = 2, 16
NW = NC * NS
CHUNK = 128
BATCH, HIST, DIM = 16384, 50, 64
TOTAL = BATCH * HIST
NCHUNK_ALL = TOTAL // CHUNK
PER_W = NCHUNK_ALL // NW

_mesh = plsc.VectorSubcoreMesh(core_axis_name="c", subcore_axis_name="s")


@functools.partial(
    pl.kernel,
    out_type=jax.ShapeDtypeStruct((TOTAL, DIM), jnp.float32),
    mesh=_mesh,
    scratch_types=[
        pltpu.VMEM((PER_W, CHUNK), jnp.int32),
        pltpu.VMEM((4, CHUNK, DIM), jnp.float32),
        pltpu.SemaphoreType.DMA((4,)),
        pltpu.SemaphoreType.DMA((4,)),
    ],
    compiler_params=pltpu.CompilerParams(use_tc_tiling_on_sc=False),
)
def _gather(idx_hbm, table_hbm, out_hbm, idx_v, rows_v, sem_g, sem_w):
    wid = lax.axis_index("s") * NC + lax.axis_index("c")
    pltpu.sync_copy(idx_hbm.at[wid], idx_v)
    base_c = wid * PER_W

    def gather_desc(t):
        b = lax.rem(t, 4)
        return pltpu.make_async_copy(
            table_hbm.at[idx_v.at[t]],
            rows_v.at[b],
            sem_g.at[b],
        )

    def write_desc(t):
        b = lax.rem(t, 4)
        return pltpu.make_async_copy(
            rows_v.at[b],
            out_hbm.at[pl.ds((base_c + t) * CHUNK, CHUNK)],
            sem_w.at[b],
        )

    gather_desc(0).start()
    gather_desc(1).start()

    def body(t):
        gather_desc(t).wait()
        write_desc(t).start()

        @pl.when(t + 2 < PER_W)
        def _():
            @pl.when(t >= 2)
            def _():
                write_desc(t - 2).wait()

            gather_desc(t + 2).start()

    pl.loop(0, PER_W)(body)
    write_desc(PER_W - 2).wait()
    write_desc(PER_W - 1).wait()


def kernel(topic_ids, W):
    idx = topic_ids.reshape(NW, PER_W, CHUNK)
    q = _gather(idx, W)
    return q.reshape(BATCH, HIST, DIM), 0
